# baseline (speedup 1.0000x reference)
import jax
import jax.numpy as jnp
from jax import lax
from jax.experimental import pallas as pl
from jax.experimental.pallas import tpu as pltpu
from jax.experimental.pallas import tpu_sc as plsc

M_ROWS = 46080
NK = 65536
NNZ = 1048576
GAMMA_N = 180
FH = 32

NUM_TILES = 16
CHUNK = 128
SUBS_PER_BLOCK = 16
BLOCK = CHUNK * SUBS_PER_BLOCK
NNZ_PER_TILE = NNZ // NUM_TILES
BLOCKS_PER_TILE = NNZ_PER_TILE // BLOCK
NCHUNK_ROWS = NNZ // CHUNK
ROWS_PER_TILE = M_ROWS // NUM_TILES
NBUF = 4
ZCOPIES = ROWS_PER_TILE // CHUNK
ZREM = ROWS_PER_TILE - ZCOPIES * CHUNK
HIMASK = -65536


def _sc_body(xs_h, cols2_h, rows_h, vals_h, out_h,
             colv, rowv, valv, bbuf, sbuf, acc, semg, sems):
    c = lax.axis_index("c")
    s = lax.axis_index("s")

    zb = sbuf.at[0]

    def z_body(i, carry):
        zb[i, pl.ds(0, 16)] = jnp.zeros((16,), jnp.float32)
        zb[i, pl.ds(16, 16)] = jnp.zeros((16,), jnp.float32)
        return carry
    lax.fori_loop(0, CHUNK, z_body, 0)

    r0z = s * ROWS_PER_TILE

    def zc_body(t, carry):
        pltpu.sync_copy(zb, acc.at[pl.ds(r0z + t * CHUNK, CHUNK)])
        return carry
    lax.fori_loop(0, ZCOPIES, zc_body, 0)
    pltpu.sync_copy(zb.at[pl.ds(0, ZREM)],
                    acc.at[pl.ds(r0z + ZCOPIES * CHUNK, ZREM)])
    plsc.subcore_barrier()

    row0 = s * (NNZ_PER_TILE // CHUNK)

    def mul_sub(k):
        b = bbuf.at[k % NBUF]
        st = sbuf.at[k % NBUF]

        @plsc.parallel_loop(0, CHUNK // 16, unroll=2)
        def _mul(i):
            vv = valv[k, pl.ds(i * 16, 16)]
            for l in range(16):
                v = vv[l]
                j = i * 16 + l
                w = b[j, :]
                lo = lax.bitcast_convert_type(w << 16, jnp.float32)
                hi = lax.bitcast_convert_type(w & HIMASK, jnp.float32)
                st[j, pl.ds(0, 16)] = lo * v
                st[j, pl.ds(16, 16)] = hi * v

    def block_body(blk, carry):
        base = row0 + blk * SUBS_PER_BLOCK
        pltpu.sync_copy(cols2_h.at[c, pl.ds(base, SUBS_PER_BLOCK)], colv)
        pltpu.sync_copy(rows_h.at[pl.ds(base, SUBS_PER_BLOCK)], rowv)
        pltpu.sync_copy(vals_h.at[pl.ds(base, SUBS_PER_BLOCK)], valv)

        gd = [None] * SUBS_PER_BLOCK
        sd = [None] * SUBS_PER_BLOCK
        for k in range(NBUF - 1):
            gd[k] = pltpu.async_copy(
                xs_h.at[colv.at[k]], bbuf.at[k % NBUF], semg.at[k % NBUF])
        for k in range(SUBS_PER_BLOCK):
            gd[k].wait()
            if k >= NBUF:
                sd[k - NBUF].wait()
            mul_sub(k)
            sd[k] = pltpu.async_copy(
                sbuf.at[k % NBUF], acc.at[rowv.at[k]], sems.at[k % NBUF],
                add=True)
            nk = k + NBUF - 1
            if nk < SUBS_PER_BLOCK and nk >= NBUF - 1:
                gd[nk] = pltpu.async_copy(
                    xs_h.at[colv.at[nk]], bbuf.at[nk % NBUF],
                    semg.at[nk % NBUF])
        for k in range(SUBS_PER_BLOCK - NBUF, SUBS_PER_BLOCK):
            sd[k].wait()
        return carry
    lax.fori_loop(0, BLOCKS_PER_TILE, block_body, 0)

    plsc.subcore_barrier()

    r0 = s * ROWS_PER_TILE
    pltpu.sync_copy(acc.at[pl.ds(r0, ROWS_PER_TILE)],
                    out_h.at[pl.ds(r0, ROWS_PER_TILE), pl.ds(c * FH, FH)])


@jax.jit
def _spmm_sc(xs, cols2, rows, vals):
    mesh = plsc.VectorSubcoreMesh(core_axis_name="c", subcore_axis_name="s")
    f = pl.kernel(
        _sc_body,
        out_type=jax.ShapeDtypeStruct((M_ROWS, 2 * FH), jnp.float32),
        mesh=mesh,
        scratch_types=[
            pltpu.VMEM((SUBS_PER_BLOCK, CHUNK), jnp.int32),
            pltpu.VMEM((SUBS_PER_BLOCK, CHUNK), jnp.int32),
            pltpu.VMEM((SUBS_PER_BLOCK, CHUNK), jnp.float32),
            pltpu.VMEM((NBUF, CHUNK, FH // 2), jnp.int32),
            pltpu.VMEM((NBUF, CHUNK, FH), jnp.float32),
            pltpu.VMEM_SHARED((M_ROWS, FH), jnp.float32),
            pltpu.SemaphoreType.DMA((NBUF,)),
            pltpu.SemaphoreType.DMA((NBUF,)),
        ],
        compiler_params=pltpu.CompilerParams(use_tc_tiling_on_sc=False),
    )
    return f(xs, cols2, rows, vals)


def kernel(x, A_indices, A_values):
    Bs, Cs, Ns, Ks = x.shape
    x2D = x.reshape(Bs * Cs, Ks * Ns)
    xsb = (x2D.reshape(2, 2, 16, NK).transpose(0, 3, 2, 1)
           .reshape(2 * NK, FH // 2, 2).astype(jnp.bfloat16))
    xs = jax.lax.bitcast_convert_type(xsb, jnp.int32)
    cols = A_indices[1]
    cols2 = jnp.stack([cols, cols + NK]).reshape(2, NCHUNK_ROWS, CHUNK)
    rows2 = A_indices[0].reshape(NCHUNK_ROWS, CHUNK)
    vals2 = A_values.reshape(NCHUNK_ROWS, CHUNK)
    proj2D = _spmm_sc(xs, cols2, rows2, vals2)
    return proj2D.reshape(Bs, Cs, GAMMA_N, -1)


# speedup vs baseline: 1.1033x; 1.1033x over previous
import jax
import jax.numpy as jnp
from jax import lax
from jax.experimental import pallas as pl
from jax.experimental.pallas import tpu as pltpu
from jax.experimental.pallas import tpu_sc as plsc

M_ROWS = 46080
NK = 65536
NNZ = 1048576
GAMMA_N = 180
FH = 32
PK = FH // 2

NUM_TILES = 16
CHUNK = 128
SUBS_PER_BLOCK = 8
BLOCK = CHUNK * SUBS_PER_BLOCK
NNZ_PER_TILE = NNZ // NUM_TILES
BLOCKS_PER_TILE = NNZ_PER_TILE // BLOCK
NCHUNK_ROWS = NNZ // CHUNK
ROWS_PER_TILE = M_ROWS // NUM_TILES
NBUF = 4
ZCOPIES = ROWS_PER_TILE // CHUNK
ZREM = ROWS_PER_TILE - ZCOPIES * CHUNK
HIMASK = -65536


def _sc_body(xs_h, cols2_h, rows_h, vals_h, out_h,
             colv, rowv, valv, bbuf, sbuf, acc, semg, sems, semi):
    c = lax.axis_index("c")
    s = lax.axis_index("s")

    zb = sbuf.at[0]

    def z_body(i, carry):
        for sl in range(NBUF):
            sbuf[sl, i, pl.ds(0, 16)] = jnp.zeros((16,), jnp.float32)
            sbuf[sl, i, pl.ds(16, 16)] = jnp.zeros((16,), jnp.float32)
        return carry
    lax.fori_loop(0, CHUNK, z_body, 0)
    for i8 in range(CHUNK // 16):
        rowv[0, 0, pl.ds(i8 * 16, 16)] = jnp.zeros((16,), jnp.int32)

    r0z = s * ROWS_PER_TILE

    def zc_body(t, carry):
        pltpu.sync_copy(zb, acc.at[pl.ds(r0z + t * CHUNK, CHUNK)])
        return carry
    lax.fori_loop(0, ZCOPIES, zc_body, 0)
    pltpu.sync_copy(zb.at[pl.ds(0, ZREM)],
                    acc.at[pl.ds(r0z + ZCOPIES * CHUNK, ZREM)])
    plsc.subcore_barrier()

    row0 = s * (NNZ_PER_TILE // CHUNK)

    def fire_idx(bdyn, p):
        base = row0 + bdyn * SUBS_PER_BLOCK
        pltpu.async_copy(cols2_h.at[c, pl.ds(base, SUBS_PER_BLOCK)],
                         colv.at[p], semi.at[p])
        pltpu.async_copy(rows_h.at[pl.ds(base, SUBS_PER_BLOCK)],
                         rowv.at[p], semi.at[p])
        pltpu.async_copy(vals_h.at[pl.ds(base, SUBS_PER_BLOCK)],
                         valv.at[p], semi.at[p])

    def wait_idx(p):
        pltpu.make_async_copy(cols2_h.at[c, pl.ds(0, SUBS_PER_BLOCK)],
                              colv.at[p], semi.at[p]).wait()
        pltpu.make_async_copy(rows_h.at[pl.ds(0, SUBS_PER_BLOCK)],
                              rowv.at[p], semi.at[p]).wait()
        pltpu.make_async_copy(vals_h.at[pl.ds(0, SUBS_PER_BLOCK)],
                              valv.at[p], semi.at[p]).wait()

    def fire_gather(p, k, slot):
        pltpu.async_copy(xs_h.at[colv.at[p].at[k]], bbuf.at[slot],
                         semg.at[slot])

    def wait_gather(slot):
        pltpu.make_async_copy(xs_h.at[colv.at[0].at[0]], bbuf.at[slot],
                              semg.at[slot]).wait()

    def fire_scatter(p, k, slot):
        pltpu.async_copy(sbuf.at[slot], acc.at[rowv.at[p].at[k]],
                         sems.at[slot], add=True)

    def wait_scatter(slot):
        pltpu.make_async_copy(sbuf.at[slot], acc.at[rowv.at[0].at[0]],
                              sems.at[slot]).wait()

    def mul_sub(p, k, slot):
        b = bbuf.at[slot]
        st = sbuf.at[slot]

        @plsc.parallel_loop(0, CHUNK // 16, unroll=2)
        def _mul(i):
            vv = valv[p, k, pl.ds(i * 16, 16)]
            for l in range(16):
                v = vv[l]
                j = i * 16 + l
                w = b[j, :]
                lo = lax.bitcast_convert_type(w << 16, jnp.float32)
                hi = lax.bitcast_convert_type(w & HIMASK, jnp.float32)
                st[j, pl.ds(0, 16)] = lo * v
                st[j, pl.ds(16, 16)] = hi * v

    def process_block(bdyn, p):
        for k in range(SUBS_PER_BLOCK):
            slot = k % NBUF
            wait_gather(slot)
            wait_scatter(slot)
            mul_sub(p, k, slot)
            fire_scatter(p, k, slot)
            if k == 4:
                fire_idx(lax.min(bdyn + 1, BLOCKS_PER_TILE - 1), 1 - p)
            nk = k + NBUF - 1
            if nk < SUBS_PER_BLOCK:
                fire_gather(p, nk, nk % NBUF)
        wait_idx(1 - p)
        for t in range(NBUF - 1):
            fire_gather(1 - p, t, (SUBS_PER_BLOCK + t) % NBUF)

    for slot in range(NBUF):
        pltpu.async_copy(sbuf.at[slot], acc.at[rowv.at[0].at[0]],
                         sems.at[slot], add=True)
    fire_idx(0, 0)
    wait_idx(0)
    for k in range(NBUF - 1):
        fire_gather(0, k, k % NBUF)

    def pair_body(i, carry):
        bdyn = 2 * i
        process_block(bdyn, 0)
        process_block(bdyn + 1, 1)
        return carry
    lax.fori_loop(0, BLOCKS_PER_TILE // 2, pair_body, 0)

    for slot in range(NBUF):
        wait_scatter(slot)
    for slot in range(NBUF - 1):
        wait_gather(slot)

    plsc.subcore_barrier()

    r0 = s * ROWS_PER_TILE
    pltpu.sync_copy(acc.at[pl.ds(r0, ROWS_PER_TILE)],
                    out_h.at[pl.ds(r0, ROWS_PER_TILE), pl.ds(c * FH, FH)])


@jax.jit
def _spmm_sc(xs, cols2, rows, vals):
    mesh = plsc.VectorSubcoreMesh(core_axis_name="c", subcore_axis_name="s")
    f = pl.kernel(
        _sc_body,
        out_type=jax.ShapeDtypeStruct((M_ROWS, 2 * FH), jnp.float32),
        mesh=mesh,
        scratch_types=[
            pltpu.VMEM((2, SUBS_PER_BLOCK, CHUNK), jnp.int32),
            pltpu.VMEM((2, SUBS_PER_BLOCK, CHUNK), jnp.int32),
            pltpu.VMEM((2, SUBS_PER_BLOCK, CHUNK), jnp.float32),
            pltpu.VMEM((NBUF, CHUNK, PK), jnp.int32),
            pltpu.VMEM((NBUF, CHUNK, FH), jnp.float32),
            pltpu.VMEM_SHARED((M_ROWS, FH), jnp.float32),
            pltpu.SemaphoreType.DMA((NBUF,)),
            pltpu.SemaphoreType.DMA((NBUF,)),
            pltpu.SemaphoreType.DMA((2,)),
        ],
        compiler_params=pltpu.CompilerParams(use_tc_tiling_on_sc=False),
    )
    return f(xs, cols2, rows, vals)


def kernel(x, A_indices, A_values):
    Bs, Cs, Ns, Ks = x.shape
    x2D = x.reshape(Bs * Cs, Ks * Ns)
    xsb = (x2D.reshape(2, 2, 16, NK).transpose(0, 3, 2, 1)
           .reshape(2 * NK, PK, 2).astype(jnp.bfloat16))
    xs = jax.lax.bitcast_convert_type(xsb, jnp.int32)
    cols = A_indices[1]
    cols2 = jnp.stack([cols, cols + NK]).reshape(2, NCHUNK_ROWS, CHUNK)
    rows2 = A_indices[0].reshape(NCHUNK_ROWS, CHUNK)
    vals2 = A_values.reshape(NCHUNK_ROWS, CHUNK)
    proj2D = _spmm_sc(xs, cols2, rows2, vals2)
    return proj2D.reshape(Bs, Cs, GAMMA_N, -1)


# speedup vs baseline: 1.5449x; 1.4002x over previous
import jax
import jax.numpy as jnp
from jax import lax
from jax.experimental import pallas as pl
from jax.experimental.pallas import tpu as pltpu
from jax.experimental.pallas import tpu_sc as plsc

M_ROWS = 46080
NK = 65536
NNZ = 1048576
GAMMA_N = 180
FH = 32

NUM_TILES = 16
CHUNK = 128
SUBS_PER_BLOCK = 16
BLOCK = CHUNK * SUBS_PER_BLOCK
NNZ_PER_TILE = NNZ // NUM_TILES
BLOCKS_PER_TILE = NNZ_PER_TILE // BLOCK
NCHUNK_ROWS = NNZ // CHUNK
ROWS_PER_TILE = M_ROWS // NUM_TILES
ZROWS = 180
NBUF = 6
NSB = 3
ZCOPIES = ROWS_PER_TILE // CHUNK
ZREM = ROWS_PER_TILE - ZCOPIES * CHUNK


def _sc_body(xs_h, cols2_h, rows_h, vals_h, out_h,
             colv, rowv, valv, gbuf, acc, semg, sems):
    c = lax.axis_index("c")
    s = lax.axis_index("s")

    zb = gbuf.at[0]

    def z_body(i, carry):
        zb[i, pl.ds(0, 16)] = jnp.zeros((16,), jnp.float32)
        zb[i, pl.ds(16, 16)] = jnp.zeros((16,), jnp.float32)
        return carry
    lax.fori_loop(0, CHUNK, z_body, 0)

    r0z = s * ROWS_PER_TILE

    def zc_body(t, carry):
        pltpu.sync_copy(zb, acc.at[pl.ds(r0z + t * CHUNK, CHUNK)])
        return carry
    lax.fori_loop(0, ZCOPIES, zc_body, 0)
    pltpu.sync_copy(zb.at[pl.ds(0, ZREM)],
                    acc.at[pl.ds(r0z + ZCOPIES * CHUNK, ZREM)])
    plsc.subcore_barrier()

    row0 = s * (NNZ_PER_TILE // CHUNK)

    def mul_sub(k):
        g = gbuf.at[k % NBUF]

        @plsc.parallel_loop(0, CHUNK // 16, unroll=4)
        def _mul(i):
            vv = valv[k, pl.ds(i * 16, 16)]
            for l in range(16):
                v = vv[l]
                j = i * 16 + l
                g0 = g[j, pl.ds(0, 16)]
                g[j, pl.ds(0, 16)] = g0 * v
                g1 = g[j, pl.ds(16, 16)]
                g[j, pl.ds(16, 16)] = g1 * v

    def block_body(blk, carry):
        base = row0 + blk * SUBS_PER_BLOCK
        pltpu.sync_copy(cols2_h.at[c, pl.ds(base, SUBS_PER_BLOCK)], colv)
        pltpu.sync_copy(rows_h.at[pl.ds(base, SUBS_PER_BLOCK)], rowv)
        pltpu.sync_copy(vals_h.at[pl.ds(base, SUBS_PER_BLOCK)], valv)

        gd = [None] * SUBS_PER_BLOCK
        sd = [None] * SUBS_PER_BLOCK
        for k in range(NBUF - 1):
            gd[k] = pltpu.async_copy(
                xs_h.at[colv.at[k]], gbuf.at[k % NBUF], semg.at[k % NBUF])
        for k in range(SUBS_PER_BLOCK):
            gd[k].wait()
            mul_sub(k)
            sd[k] = pltpu.async_copy(
                gbuf.at[k % NBUF], acc.at[rowv.at[k]], sems.at[k % NBUF],
                add=True)
            nk = k + NBUF - 1
            if nk < SUBS_PER_BLOCK and nk >= NBUF - 1:
                if nk - NBUF >= 0:
                    sd[nk - NBUF].wait()
                gd[nk] = pltpu.async_copy(
                    xs_h.at[colv.at[nk]], gbuf.at[nk % NBUF],
                    semg.at[nk % NBUF])
        for k in range(SUBS_PER_BLOCK - NBUF, SUBS_PER_BLOCK):
            sd[k].wait()
        return carry
    lax.fori_loop(0, BLOCKS_PER_TILE, block_body, 0)

    plsc.subcore_barrier()

    r0 = s * ROWS_PER_TILE
    pltpu.sync_copy(acc.at[pl.ds(r0, ROWS_PER_TILE)],
                    out_h.at[pl.ds(r0, ROWS_PER_TILE), pl.ds(c * FH, FH)])


@jax.jit
def _spmm_sc(xs, cols2, rows, vals):
    mesh = plsc.VectorSubcoreMesh(core_axis_name="c", subcore_axis_name="s")
    f = pl.kernel(
        _sc_body,
        out_type=jax.ShapeDtypeStruct((M_ROWS, 2 * FH), jnp.float32),
        mesh=mesh,
        scratch_types=[
            pltpu.VMEM((SUBS_PER_BLOCK, CHUNK), jnp.int32),
            pltpu.VMEM((SUBS_PER_BLOCK, CHUNK), jnp.int32),
            pltpu.VMEM((SUBS_PER_BLOCK, CHUNK), jnp.float32),
            pltpu.VMEM((NBUF, CHUNK, FH), jnp.float32),
            pltpu.VMEM_SHARED((M_ROWS, FH), jnp.float32),
            pltpu.SemaphoreType.DMA((NBUF,)),
            pltpu.SemaphoreType.DMA((NBUF,)),
        ],
        compiler_params=pltpu.CompilerParams(use_tc_tiling_on_sc=False),
    )
    return f(xs, cols2, rows, vals)


def kernel(x, A_indices, A_values):
    Bs, Cs, Ns, Ks = x.shape
    x2D = x.reshape(Bs * Cs, Ks * Ns)
    xs = x2D.reshape(2, FH, NK).transpose(0, 2, 1).reshape(2 * NK, FH)
    cols = A_indices[1]
    cols2 = jnp.stack([cols, cols + NK]).reshape(2, NCHUNK_ROWS, CHUNK)
    rows2 = A_indices[0].reshape(NCHUNK_ROWS, CHUNK)
    vals2 = A_values.reshape(NCHUNK_ROWS, CHUNK)
    proj2D = _spmm_sc(xs, cols2, rows2, vals2)
    return proj2D.reshape(Bs, Cs, GAMMA_N, -1)
